# Initial kernel scaffold; baseline (speedup 1.0000x reference)
import jax
import jax.numpy as jnp
from jax.experimental import pallas as pl


def kernel(x, edge_index, edge_weight, ent_emb, rel_trans):
    raise NotImplementedError("write your pallas kernel here")


# speedup vs baseline: 1.2847x; 1.2847x over previous
import functools

import jax
import jax.numpy as jnp
from jax import lax
from jax.experimental import pallas as pl
from jax.experimental.pallas import tpu as pltpu
from jax.experimental.pallas import tpu_sc as plsc

N = 10000
R = 4
E = 150000
D = 128

NUM_TILES = 16
EPT = E // NUM_TILES
CHUNK = 128
NCH = 80
EPT_PAD = NCH * CHUNK
N_PAD = 10240
ROWS_PT = N_PAD // NUM_TILES
ZR = 128
ZB = 16
NST = 2
CH_ST = NCH // NST


def _splat_lane(vec, k):
    idx = jnp.full((16, 1), k, jnp.int32)
    dnums = lax.GatherDimensionNumbers(
        offset_dims=(), collapsed_slice_dims=(0,), start_index_map=(0,))
    return lax.gather(vec, idx, dnums, slice_sizes=(1,),
                      mode=lax.GatherScatterMode.PROMISE_IN_BOUNDS)


def _sc_aggregate_body(emb_hbm, src_hbm, dst_hbm, w_hbm, out_hbm,
                       src_v, dst_v, w_v, rows_v, zeros_v, acc_sh, sem):
    c = lax.axis_index("c")
    s = lax.axis_index("s")
    zvec = jnp.zeros((16,), jnp.float32)

    def zero_body(i, _):
        for k in range(D // 16):
            zeros_v[i, pl.ds(16 * k, 16)] = zvec
        return 0
    lax.fori_loop(0, ZB, zero_body, 0)

    for rl in range(2):
        r = 2 * c + rl
        def zero_acc(q, _):
            pltpu.sync_copy(zeros_v, acc_sh.at[pl.ds(s * ROWS_PT + q * ZB, ZB)])
            return 0
        lax.fori_loop(0, ROWS_PT // ZB, zero_acc, 0)
        plsc.subcore_barrier()

        def stage_body(h, _):
            pltpu.sync_copy(src_hbm.at[r, s, pl.ds(h * CH_ST, CH_ST)], src_v)
            pltpu.sync_copy(dst_hbm.at[r, s, pl.ds(h * CH_ST, CH_ST)], dst_v)
            pltpu.sync_copy(w_hbm.at[r, s, pl.ds(h * CH_ST, CH_ST)], w_v)
            lax.fori_loop(0, CH_ST, chunk_body, 0)
            return 0

        def chunk_body(j, _):
            pltpu.async_copy(emb_hbm.at[src_v.at[j]], rows_v, sem).wait()

            def group_body(g, _):
                w16 = w_v[j, pl.ds(16 * g, 16)]
                for k in range(16):
                    wsp = _splat_lane(w16, k)
                    i = g * 16 + k
                    for q in range(D // 16):
                        sl = pl.ds(16 * q, 16)
                        rows_v[i, sl] = rows_v[i, sl] * wsp
                return 0
            lax.fori_loop(0, CHUNK // 16, group_body, 0)

            pltpu.sync_copy(rows_v, acc_sh.at[dst_v.at[j]], add=True)
            return 0
        lax.fori_loop(0, NST, stage_body, 0)
        plsc.subcore_barrier()

        for q in range(ROWS_PT // ZR):
            base = s * ROWS_PT + q * ZR
            pltpu.sync_copy(acc_sh.at[pl.ds(base, ZR)],
                            out_hbm.at[r, pl.ds(base, ZR)])
        plsc.subcore_barrier()


@functools.cache
def _get_sc_aggregate():
    return pl.kernel(
        _sc_aggregate_body,
        out_type=jax.ShapeDtypeStruct((R, N_PAD, D), jnp.float32),
        mesh=plsc.VectorSubcoreMesh(core_axis_name="c", subcore_axis_name="s",
                                    num_cores=2, num_subcores=NUM_TILES),
        scratch_types=[
            pltpu.VMEM((CH_ST, CHUNK), jnp.int32),
            pltpu.VMEM((CH_ST, CHUNK), jnp.int32),
            pltpu.VMEM((CH_ST, CHUNK), jnp.float32),
            pltpu.VMEM((CHUNK, D), jnp.float32),
            pltpu.VMEM((ZB, D), jnp.float32),
            pltpu.VMEM_SHARED((N_PAD, D), jnp.float32),
            pltpu.SemaphoreType.DMA,
        ],
    )


def _matmul_body(x_ref, w_ref, o_ref):
    o_ref[...] = jnp.dot(x_ref[...], w_ref[...],
                         preferred_element_type=jnp.float32)


def _transform_body(acc_ref, w_ref, o_ref, *, last):
    t = jnp.zeros(o_ref.shape, jnp.float32)
    for r in range(R):
        t = t + lax.dot_general(
            acc_ref[r], w_ref[r],
            dimension_numbers=(((1,), (1,)), ((), ())),
            preferred_element_type=jnp.float32)
    t = jnp.maximum(t, 0.0)
    if last:
        nrm = jnp.sqrt(jnp.sum(t * t, axis=1, keepdims=True))
        t = t / jnp.maximum(nrm, 1e-12)
    o_ref[...] = t


_BN = 1000


def _transform(acc, w, last):
    body = functools.partial(_transform_body, last=last)
    return pl.pallas_call(
        body,
        grid=(N // _BN,),
        in_specs=[
            pl.BlockSpec((R, _BN, D), lambda i: (0, i, 0)),
            pl.BlockSpec((R, D, D), lambda i: (0, 0, 0)),
        ],
        out_specs=pl.BlockSpec((_BN, D), lambda i: (i, 0)),
        out_shape=jax.ShapeDtypeStruct((N, D), jnp.float32),
    )(acc, w)


def kernel(x, edge_index, edge_weight, ent_emb, rel_trans):
    src = edge_index[:, 1, :].reshape(R, NUM_TILES, EPT)
    dst = edge_index[:, 0, :].reshape(R, NUM_TILES, EPT)
    w = edge_weight.reshape(R, NUM_TILES, EPT)
    pad = ((0, 0), (0, 0), (0, EPT_PAD - EPT))
    src = jnp.pad(src, pad).reshape(R, NUM_TILES, NCH, CHUNK)
    dst = jnp.pad(dst, pad).reshape(R, NUM_TILES, NCH, CHUNK)
    w = jnp.pad(w, pad).reshape(R, NUM_TILES, NCH, CHUNK)

    emb = pl.pallas_call(
        _matmul_body,
        grid=(N // _BN,),
        in_specs=[
            pl.BlockSpec((_BN, D), lambda i: (i, 0)),
            pl.BlockSpec((D, D), lambda i: (0, 0)),
        ],
        out_specs=pl.BlockSpec((_BN, D), lambda i: (i, 0)),
        out_shape=jax.ShapeDtypeStruct((N, D), jnp.float32),
    )(x, ent_emb)

    n_layers = rel_trans.shape[0]
    for l in range(n_layers):
        acc = _get_sc_aggregate()(emb, src, dst, w)
        emb = _transform(acc, rel_trans[l], last=(l == n_layers - 1))
    return emb


# speedup vs baseline: 1.3583x; 1.0573x over previous
import functools

import jax
import jax.numpy as jnp
from jax import lax
from jax.experimental import pallas as pl
from jax.experimental.pallas import tpu as pltpu
from jax.experimental.pallas import tpu_sc as plsc

N = 10000
R = 4
E = 150000
D = 128

NUM_TILES = 16
EPT = E // NUM_TILES
CHUNK = 128
NCH = 80
EPT_PAD = NCH * CHUNK
N_PAD = 10240
ROWS_PT = N_PAD // NUM_TILES
ZR = 128
ZB = 8
NST = 5
CH_ST = NCH // NST


def _splat_lane(vec, k):
    idx = jnp.full((16, 1), k, jnp.int32)
    dnums = lax.GatherDimensionNumbers(
        offset_dims=(), collapsed_slice_dims=(0,), start_index_map=(0,))
    return lax.gather(vec, idx, dnums, slice_sizes=(1,),
                      mode=lax.GatherScatterMode.PROMISE_IN_BOUNDS)


def _sc_aggregate_body(emb_hbm, src_hbm, dst_hbm, w_hbm, out_hbm,
                       src_v, dst_v, w_v, rows0_v, rows1_v, zeros_v, acc_sh,
                       gsem0, gsem1):
    c = lax.axis_index("c")
    s = lax.axis_index("s")
    zvec = jnp.zeros((16,), jnp.float32)

    def zero_body(i, _):
        for k in range(D // 16):
            zeros_v[i, pl.ds(16 * k, 16)] = zvec
        return 0
    lax.fori_loop(0, ZB, zero_body, 0)

    def scale_rows(rows_v, j):
        def group_body(g, _):
            w16 = w_v[j, pl.ds(16 * g, 16)]
            for k in range(16):
                wsp = _splat_lane(w16, k)
                i = g * 16 + k
                for q in range(D // 16):
                    sl = pl.ds(16 * q, 16)
                    rows_v[i, sl] = rows_v[i, sl] * wsp
            return 0
        lax.fori_loop(0, CHUNK // 16, group_body, 0)

    for rl in range(2):
        r = 2 * c + rl
        def zero_acc(q, _):
            pltpu.sync_copy(zeros_v, acc_sh.at[pl.ds(s * ROWS_PT + q * ZB, ZB)])
            return 0
        lax.fori_loop(0, ROWS_PT // ZB, zero_acc, 0)
        plsc.subcore_barrier()

        def stage_body(h, _):
            pltpu.sync_copy(src_hbm.at[r, s, pl.ds(h * CH_ST, CH_ST)], src_v)
            pltpu.sync_copy(dst_hbm.at[r, s, pl.ds(h * CH_ST, CH_ST)], dst_v)
            pltpu.sync_copy(w_hbm.at[r, s, pl.ds(h * CH_ST, CH_ST)], w_v)

            def pair_body(t, _):
                j0 = 2 * t
                j1 = 2 * t + 1

                @pl.when(t == 0)
                def _():
                    pltpu.async_copy(emb_hbm.at[src_v.at[j0]], rows0_v, gsem0)

                pltpu.async_copy(emb_hbm.at[src_v.at[j1]], rows1_v, gsem1)
                pltpu.make_async_copy(emb_hbm.at[src_v.at[j0]], rows0_v,
                                      gsem0).wait()
                scale_rows(rows0_v, j0)
                pltpu.sync_copy(rows0_v, acc_sh.at[dst_v.at[j0]], add=True)

                @pl.when(t + 1 < CH_ST // 2)
                def _():
                    pltpu.async_copy(emb_hbm.at[src_v.at[j0 + 2]], rows0_v,
                                     gsem0)

                pltpu.make_async_copy(emb_hbm.at[src_v.at[j1]], rows1_v,
                                      gsem1).wait()
                scale_rows(rows1_v, j1)
                pltpu.sync_copy(rows1_v, acc_sh.at[dst_v.at[j1]], add=True)
                return 0
            lax.fori_loop(0, CH_ST // 2, pair_body, 0)
            return 0
        lax.fori_loop(0, NST, stage_body, 0)
        plsc.subcore_barrier()

        for q in range(ROWS_PT // ZR):
            base = s * ROWS_PT + q * ZR
            pltpu.sync_copy(acc_sh.at[pl.ds(base, ZR)],
                            out_hbm.at[r, pl.ds(base, ZR)])
        plsc.subcore_barrier()


@functools.cache
def _get_sc_aggregate():
    return pl.kernel(
        _sc_aggregate_body,
        out_type=jax.ShapeDtypeStruct((R, N_PAD, D), jnp.float32),
        mesh=plsc.VectorSubcoreMesh(core_axis_name="c", subcore_axis_name="s",
                                    num_cores=2, num_subcores=NUM_TILES),
        scratch_types=[
            pltpu.VMEM((CH_ST, CHUNK), jnp.int32),
            pltpu.VMEM((CH_ST, CHUNK), jnp.int32),
            pltpu.VMEM((CH_ST, CHUNK), jnp.float32),
            pltpu.VMEM((CHUNK, D), jnp.float32),
            pltpu.VMEM((CHUNK, D), jnp.float32),
            pltpu.VMEM((ZB, D), jnp.float32),
            pltpu.VMEM_SHARED((N_PAD, D), jnp.float32),
            pltpu.SemaphoreType.DMA,
            pltpu.SemaphoreType.DMA,
        ],
    )


def _matmul_body(x_ref, w_ref, o_ref):
    o_ref[...] = jnp.dot(x_ref[...], w_ref[...],
                         preferred_element_type=jnp.float32)


def _transform_body(acc_ref, w_ref, o_ref, *, last):
    t = jnp.zeros(o_ref.shape, jnp.float32)
    for r in range(R):
        t = t + lax.dot_general(
            acc_ref[r], w_ref[r],
            dimension_numbers=(((1,), (1,)), ((), ())),
            preferred_element_type=jnp.float32)
    t = jnp.maximum(t, 0.0)
    if last:
        nrm = jnp.sqrt(jnp.sum(t * t, axis=1, keepdims=True))
        t = t / jnp.maximum(nrm, 1e-12)
    o_ref[...] = t


_BN = 1000


def _transform(acc, w, last):
    body = functools.partial(_transform_body, last=last)
    return pl.pallas_call(
        body,
        grid=(N // _BN,),
        in_specs=[
            pl.BlockSpec((R, _BN, D), lambda i: (0, i, 0)),
            pl.BlockSpec((R, D, D), lambda i: (0, 0, 0)),
        ],
        out_specs=pl.BlockSpec((_BN, D), lambda i: (i, 0)),
        out_shape=jax.ShapeDtypeStruct((N, D), jnp.float32),
    )(acc, w)


def kernel(x, edge_index, edge_weight, ent_emb, rel_trans):
    src = edge_index[:, 1, :].reshape(R, NUM_TILES, EPT)
    dst = edge_index[:, 0, :].reshape(R, NUM_TILES, EPT)
    w = edge_weight.reshape(R, NUM_TILES, EPT)
    pad = ((0, 0), (0, 0), (0, EPT_PAD - EPT))
    src = jnp.pad(src, pad).reshape(R, NUM_TILES, NCH, CHUNK)
    dst = jnp.pad(dst, pad).reshape(R, NUM_TILES, NCH, CHUNK)
    w = jnp.pad(w, pad).reshape(R, NUM_TILES, NCH, CHUNK)

    emb = pl.pallas_call(
        _matmul_body,
        grid=(N // _BN,),
        in_specs=[
            pl.BlockSpec((_BN, D), lambda i: (i, 0)),
            pl.BlockSpec((D, D), lambda i: (0, 0)),
        ],
        out_specs=pl.BlockSpec((_BN, D), lambda i: (i, 0)),
        out_shape=jax.ShapeDtypeStruct((N, D), jnp.float32),
    )(x, ent_emb)

    n_layers = rel_trans.shape[0]
    for l in range(n_layers):
        acc = _get_sc_aggregate()(emb, src, dst, w)
        emb = _transform(acc, rel_trans[l], last=(l == n_layers - 1))
    return emb


# speedup vs baseline: 2.1933x; 1.6147x over previous
import functools

import jax
import jax.numpy as jnp
from jax import lax
from jax.experimental import pallas as pl
from jax.experimental.pallas import tpu as pltpu
from jax.experimental.pallas import tpu_sc as plsc

N = 10000
R = 4
E = 150000
D = 128

NUM_TILES = 16
EPT = E // NUM_TILES
CHUNK = 128
NCH = 80
EPT_PAD = NCH * CHUNK
N_PAD = 10240
ROWS_PT = N_PAD // NUM_TILES
ZR = 128
ZB = 8
SB = 8
NBLK = NCH // SB
NPASS = 3
TP = 3456
TROWS_PT = TP // NUM_TILES
N_TAB = NPASS * TP


def _splat_lane(vec, k):
    idx = jnp.full((16, 1), k, jnp.int32)
    dnums = lax.GatherDimensionNumbers(
        offset_dims=(), collapsed_slice_dims=(0,), start_index_map=(0,))
    return lax.gather(vec, idx, dnums, slice_sizes=(1,),
                      mode=lax.GatherScatterMode.PROMISE_IN_BOUNDS)


def _sc_aggregate_body(emb_hbm, src_hbm, dst_hbm, w_hbm, out_hbm,
                       src_v, dst_v, w_v, rows_v, zeros_v, tab_sh, acc_sh,
                       gsem):
    c = lax.axis_index("c")
    s = lax.axis_index("s")
    zvec = jnp.zeros((16,), jnp.float32)

    def zero_body(i, _):
        for k in range(D // 16):
            zeros_v[i, pl.ds(16 * k, 16)] = zvec
        return 0
    lax.fori_loop(0, ZB, zero_body, 0)

    def scale_rows(j):
        def group_body(g, _):
            w16 = w_v[j, pl.ds(16 * g, 16)]
            for k in range(16):
                wsp = _splat_lane(w16, k)
                i = g * 16 + k
                for q in range(D // 16):
                    sl = pl.ds(16 * q, 16)
                    rows_v[i, sl] = rows_v[i, sl] * wsp
            return 0
        lax.fori_loop(0, CHUNK // 16, group_body, 0)

    for rl in range(2):
        r = 2 * c + rl
        def zero_acc(q, _):
            pltpu.sync_copy(zeros_v, acc_sh.at[pl.ds(s * ROWS_PT + q * ZB, ZB)])
            return 0
        lax.fori_loop(0, ROWS_PT // ZB, zero_acc, 0)
        plsc.subcore_barrier()

        for p in range(NPASS):
            base = s * TROWS_PT
            pltpu.sync_copy(emb_hbm.at[pl.ds(p * TP + base, TROWS_PT)],
                            tab_sh.at[pl.ds(base, TROWS_PT)])
            plsc.subcore_barrier()

            def block_body(h, _):
                blk = pl.ds(h * SB, SB)
                pltpu.sync_copy(src_hbm.at[p, r, s, blk], src_v)
                pltpu.sync_copy(dst_hbm.at[r, s, blk], dst_v)
                pltpu.sync_copy(w_hbm.at[p, r, s, blk], w_v)

                def chunk_body(j, _):
                    pltpu.async_copy(tab_sh.at[src_v.at[j]], rows_v,
                                     gsem).wait()
                    scale_rows(j)
                    pltpu.sync_copy(rows_v, acc_sh.at[dst_v.at[j]], add=True)
                    return 0
                lax.fori_loop(0, SB, chunk_body, 0)
                return 0
            lax.fori_loop(0, NBLK, block_body, 0)
            plsc.subcore_barrier()

        for q in range(ROWS_PT // ZR):
            base = s * ROWS_PT + q * ZR
            pltpu.sync_copy(acc_sh.at[pl.ds(base, ZR)],
                            out_hbm.at[r, pl.ds(base, ZR)])
        plsc.subcore_barrier()


@functools.cache
def _get_sc_aggregate():
    return pl.kernel(
        _sc_aggregate_body,
        out_type=jax.ShapeDtypeStruct((R, N_PAD, D), jnp.float32),
        mesh=plsc.VectorSubcoreMesh(core_axis_name="c", subcore_axis_name="s",
                                    num_cores=2, num_subcores=NUM_TILES),
        scratch_types=[
            pltpu.VMEM((SB, CHUNK), jnp.int32),
            pltpu.VMEM((SB, CHUNK), jnp.int32),
            pltpu.VMEM((SB, CHUNK), jnp.float32),
            pltpu.VMEM((CHUNK, D), jnp.float32),
            pltpu.VMEM((ZB, D), jnp.float32),
            pltpu.VMEM_SHARED((TP, D), jnp.float32),
            pltpu.VMEM_SHARED((N_PAD, D), jnp.float32),
            pltpu.SemaphoreType.DMA,
        ],
    )


def _matmul_body(x_ref, w_ref, o_ref):
    o_ref[...] = jnp.dot(x_ref[...], w_ref[...],
                         preferred_element_type=jnp.float32)


def _transform_body(acc_ref, w_ref, o_ref, *, last):
    t = jnp.zeros(o_ref.shape, jnp.float32)
    for r in range(R):
        t = t + lax.dot_general(
            acc_ref[r], w_ref[r],
            dimension_numbers=(((1,), (1,)), ((), ())),
            preferred_element_type=jnp.float32)
    t = jnp.maximum(t, 0.0)
    if last:
        nrm = jnp.sqrt(jnp.sum(t * t, axis=1, keepdims=True))
        t = t / jnp.maximum(nrm, 1e-12)
    o_ref[...] = t


_BN = 1000


def _transform(acc, w, last):
    body = functools.partial(_transform_body, last=last)
    return pl.pallas_call(
        body,
        grid=(N // _BN,),
        in_specs=[
            pl.BlockSpec((R, _BN, D), lambda i: (0, i, 0)),
            pl.BlockSpec((R, D, D), lambda i: (0, 0, 0)),
        ],
        out_specs=pl.BlockSpec((_BN, D), lambda i: (i, 0)),
        out_shape=jax.ShapeDtypeStruct((N, D), jnp.float32),
    )(acc, w)


def kernel(x, edge_index, edge_weight, ent_emb, rel_trans):
    src = edge_index[:, 1, :].reshape(R, NUM_TILES, EPT)
    dst = edge_index[:, 0, :].reshape(R, NUM_TILES, EPT)
    w = edge_weight.reshape(R, NUM_TILES, EPT)
    pad = ((0, 0), (0, 0), (0, EPT_PAD - EPT))
    src = jnp.pad(src, pad).reshape(R, NUM_TILES, NCH, CHUNK)
    dst = jnp.pad(dst, pad).reshape(R, NUM_TILES, NCH, CHUNK)
    w = jnp.pad(w, pad).reshape(R, NUM_TILES, NCH, CHUNK)

    lo = (jnp.arange(NPASS, dtype=jnp.int32) * TP)[:, None, None, None, None]
    src_loc = jnp.clip(src[None] - lo, 0, TP - 1).astype(jnp.int32)
    in_range = (src[None] >= lo) & (src[None] < lo + TP)
    w_pass = jnp.where(in_range, w[None], 0.0)

    emb = pl.pallas_call(
        _matmul_body,
        grid=(N // _BN,),
        in_specs=[
            pl.BlockSpec((_BN, D), lambda i: (i, 0)),
            pl.BlockSpec((D, D), lambda i: (0, 0)),
        ],
        out_specs=pl.BlockSpec((_BN, D), lambda i: (i, 0)),
        out_shape=jax.ShapeDtypeStruct((N, D), jnp.float32),
    )(x, ent_emb)

    n_layers = rel_trans.shape[0]
    for l in range(n_layers):
        emb_pad = jnp.pad(emb, ((0, N_TAB - N), (0, 0)))
        acc = _get_sc_aggregate()(emb_pad, src_loc, dst, w_pass)
        emb = _transform(acc, rel_trans[l], last=(l == n_layers - 1))
    return emb


# speedup vs baseline: 2.2352x; 1.0191x over previous
import functools

import jax
import jax.numpy as jnp
from jax import lax
from jax.experimental import pallas as pl
from jax.experimental.pallas import tpu as pltpu
from jax.experimental.pallas import tpu_sc as plsc

N = 10000
R = 4
E = 150000
D = 128

NUM_TILES = 16
EPT = E // NUM_TILES
CHUNK = 128
NCH = 80
EPT_PAD = NCH * CHUNK
N_PAD = 10240
ROWS_PT = N_PAD // NUM_TILES
ZR = 128
ZB = 8
SB = 8
NBLK = NCH // SB
NPASS = 3
TP = 3456
TROWS_PT = TP // NUM_TILES
N_TAB = NPASS * TP


def _splat_lane(vec, k):
    idx = jnp.full((16, 1), k, jnp.int32)
    dnums = lax.GatherDimensionNumbers(
        offset_dims=(), collapsed_slice_dims=(0,), start_index_map=(0,))
    return lax.gather(vec, idx, dnums, slice_sizes=(1,),
                      mode=lax.GatherScatterMode.PROMISE_IN_BOUNDS)


def _sc_aggregate_body(emb_hbm, src_hbm, dst_hbm, w_hbm, out_hbm,
                       src_v, dst_v, w_v, rows_v, zeros_v, tab_sh, acc_sh,
                       gsem):
    c = lax.axis_index("c")
    s = lax.axis_index("s")
    zvec = jnp.zeros((16,), jnp.float32)

    def zero_body(i, _):
        for k in range(D // 16):
            zeros_v[i, pl.ds(16 * k, 16)] = zvec
        return 0
    lax.fori_loop(0, ZB, zero_body, 0)

    def scale_rows(j):
        def group_body(g2, _):
            for gg in range(2):
                g = g2 * 2 + gg
                w16 = w_v[j, pl.ds(16 * g, 16)]
                for k in range(16):
                    wsp = _splat_lane(w16, k)
                    i = g * 16 + k
                    for q in range(D // 16):
                        sl = pl.ds(16 * q, 16)
                        rows_v[i, sl] = rows_v[i, sl] * wsp
            return 0
        lax.fori_loop(0, CHUNK // 32, group_body, 0)

    for rl in range(2):
        r = 2 * c + rl
        def zero_acc(q, _):
            pltpu.sync_copy(zeros_v, acc_sh.at[pl.ds(s * ROWS_PT + q * ZB, ZB)])
            return 0
        lax.fori_loop(0, ROWS_PT // ZB, zero_acc, 0)
        plsc.subcore_barrier()

        for p in range(NPASS):
            base = s * TROWS_PT
            pltpu.sync_copy(emb_hbm.at[pl.ds(p * TP + base, TROWS_PT)],
                            tab_sh.at[pl.ds(base, TROWS_PT)])
            plsc.subcore_barrier()

            def block_body(h, _):
                blk = pl.ds(h * SB, SB)
                pltpu.async_copy(src_hbm.at[p, r, s, blk], src_v, gsem)
                pltpu.async_copy(dst_hbm.at[r, s, blk], dst_v, gsem)
                pltpu.async_copy(w_hbm.at[p, r, s, blk], w_v, gsem)
                pltpu.make_async_copy(src_hbm.at[p, r, s, blk], src_v,
                                      gsem).wait()
                pltpu.make_async_copy(dst_hbm.at[r, s, blk], dst_v,
                                      gsem).wait()
                pltpu.make_async_copy(w_hbm.at[p, r, s, blk], w_v,
                                      gsem).wait()

                def chunk_body(j, _):
                    pltpu.async_copy(tab_sh.at[src_v.at[j]], rows_v,
                                     gsem).wait()
                    scale_rows(j)
                    pltpu.sync_copy(rows_v, acc_sh.at[dst_v.at[j]], add=True)
                    return 0
                lax.fori_loop(0, SB, chunk_body, 0)
                return 0
            lax.fori_loop(0, NBLK, block_body, 0)
            plsc.subcore_barrier()

        for q in range(ROWS_PT // ZR):
            base = s * ROWS_PT + q * ZR
            pltpu.sync_copy(acc_sh.at[pl.ds(base, ZR)],
                            out_hbm.at[r, pl.ds(base, ZR)])
        plsc.subcore_barrier()


@functools.cache
def _get_sc_aggregate():
    return pl.kernel(
        _sc_aggregate_body,
        out_type=jax.ShapeDtypeStruct((R, N_PAD, D), jnp.float32),
        mesh=plsc.VectorSubcoreMesh(core_axis_name="c", subcore_axis_name="s",
                                    num_cores=2, num_subcores=NUM_TILES),
        scratch_types=[
            pltpu.VMEM((SB, CHUNK), jnp.int32),
            pltpu.VMEM((SB, CHUNK), jnp.int32),
            pltpu.VMEM((SB, CHUNK), jnp.float32),
            pltpu.VMEM((CHUNK, D), jnp.float32),
            pltpu.VMEM((ZB, D), jnp.float32),
            pltpu.VMEM_SHARED((TP, D), jnp.float32),
            pltpu.VMEM_SHARED((N_PAD, D), jnp.float32),
            pltpu.SemaphoreType.DMA,
        ],
    )


def _matmul_body(x_ref, w_ref, o_ref):
    o_ref[...] = jnp.dot(x_ref[...], w_ref[...],
                         preferred_element_type=jnp.float32)


def _transform_body(acc_ref, w_ref, o_ref, *, last):
    t = jnp.zeros(o_ref.shape, jnp.float32)
    for r in range(R):
        t = t + lax.dot_general(
            acc_ref[r], w_ref[r],
            dimension_numbers=(((1,), (1,)), ((), ())),
            preferred_element_type=jnp.float32)
    t = jnp.maximum(t, 0.0)
    if last:
        nrm = jnp.sqrt(jnp.sum(t * t, axis=1, keepdims=True))
        t = t / jnp.maximum(nrm, 1e-12)
    o_ref[...] = t


_BN = 1000


def _transform(acc, w, last):
    body = functools.partial(_transform_body, last=last)
    return pl.pallas_call(
        body,
        grid=(N // _BN,),
        in_specs=[
            pl.BlockSpec((R, _BN, D), lambda i: (0, i, 0)),
            pl.BlockSpec((R, D, D), lambda i: (0, 0, 0)),
        ],
        out_specs=pl.BlockSpec((_BN, D), lambda i: (i, 0)),
        out_shape=jax.ShapeDtypeStruct((N, D), jnp.float32),
    )(acc, w)


def kernel(x, edge_index, edge_weight, ent_emb, rel_trans):
    src = edge_index[:, 1, :].reshape(R, NUM_TILES, EPT)
    dst = edge_index[:, 0, :].reshape(R, NUM_TILES, EPT)
    w = edge_weight.reshape(R, NUM_TILES, EPT)
    pad = ((0, 0), (0, 0), (0, EPT_PAD - EPT))
    src = jnp.pad(src, pad).reshape(R, NUM_TILES, NCH, CHUNK)
    dst = jnp.pad(dst, pad).reshape(R, NUM_TILES, NCH, CHUNK)
    w = jnp.pad(w, pad).reshape(R, NUM_TILES, NCH, CHUNK)

    lo = (jnp.arange(NPASS, dtype=jnp.int32) * TP)[:, None, None, None, None]
    src_loc = jnp.clip(src[None] - lo, 0, TP - 1).astype(jnp.int32)
    in_range = (src[None] >= lo) & (src[None] < lo + TP)
    w_pass = jnp.where(in_range, w[None], 0.0)

    emb = pl.pallas_call(
        _matmul_body,
        grid=(N // _BN,),
        in_specs=[
            pl.BlockSpec((_BN, D), lambda i: (i, 0)),
            pl.BlockSpec((D, D), lambda i: (0, 0)),
        ],
        out_specs=pl.BlockSpec((_BN, D), lambda i: (i, 0)),
        out_shape=jax.ShapeDtypeStruct((N, D), jnp.float32),
    )(x, ent_emb)

    n_layers = rel_trans.shape[0]
    for l in range(n_layers):
        emb_pad = jnp.pad(emb, ((0, N_TAB - N), (0, 0)))
        acc = _get_sc_aggregate()(emb_pad, src_loc, dst, w_pass)
        emb = _transform(acc, rel_trans[l], last=(l == n_layers - 1))
    return emb


# speedup vs baseline: 3.2506x; 1.4543x over previous
import functools

import jax
import jax.numpy as jnp
from jax import lax
from jax.experimental import pallas as pl
from jax.experimental.pallas import tpu as pltpu
from jax.experimental.pallas import tpu_sc as plsc

N = 10000
R = 4
E = 150000
D = 128

NUM_TILES = 16
EPT = E // NUM_TILES
CHUNK = 128
NCH = 80
EPT_PAD = NCH * CHUNK
N_PAD = 10240
ROWS_PT = N_PAD // NUM_TILES
ZR = 128
ZB = 8
SB = 8
NBLK = NCH // SB
NPASS = 2
TROWS_PT = 168
TAB_P = NUM_TILES * TROWS_PT
TP = 2 * TAB_P
N_TAB = NPASS * TP


def _splat_lane(vec, k):
    idx = jnp.full((16, 1), k, jnp.int32)
    dnums = lax.GatherDimensionNumbers(
        offset_dims=(), collapsed_slice_dims=(0,), start_index_map=(0,))
    return lax.gather(vec, idx, dnums, slice_sizes=(1,),
                      mode=lax.GatherScatterMode.PROMISE_IN_BOUNDS)


def _sc_aggregate_body(tab_hbm, src_hbm, dst_hbm, w_hbm, sh_hbm, out_hbm,
                       src_v, dst_v, w_v, sh_v, rows_v, zeros_v, tab_sh,
                       acc_sh, gsem):
    c = lax.axis_index("c")
    s = lax.axis_index("s")
    zvec = jnp.zeros((16,), jnp.float32)

    def zero_body(i, _):
        for k in range(D // 16):
            zeros_v[i, pl.ds(16 * k, 16)] = zvec
        return 0
    lax.fori_loop(0, ZB, zero_body, 0)

    def scale_rows(j):
        def group_body(g2, _):
            for gg in range(2):
                g = g2 * 2 + gg
                w16 = w_v[j, pl.ds(16 * g, 16)]
                sh16 = sh_v[j, pl.ds(16 * g, 16)]
                for k in range(16):
                    wsp = _splat_lane(w16, k)
                    shsp = _splat_lane(sh16, k)
                    i = g * 16 + k
                    for q in range(D // 16):
                        sl = pl.ds(16 * q, 16)
                        bits = lax.bitcast_convert_type(rows_v[i, sl], jnp.int32)
                        bits = lax.shift_left(
                            lax.shift_right_logical(bits, shsp), 16)
                        rows_v[i, sl] = lax.bitcast_convert_type(bits, jnp.float32) * wsp
            return 0
        lax.fori_loop(0, CHUNK // 32, group_body, 0)

    for rl in range(2):
        r = 2 * c + rl
        def zero_acc(q, _):
            pltpu.sync_copy(zeros_v, acc_sh.at[pl.ds(s * ROWS_PT + q * ZB, ZB)])
            return 0
        lax.fori_loop(0, ROWS_PT // ZB, zero_acc, 0)
        plsc.subcore_barrier()

        for p in range(NPASS):
            base = s * TROWS_PT
            pltpu.sync_copy(tab_hbm.at[pl.ds(p * TAB_P + base, TROWS_PT)],
                            tab_sh.at[pl.ds(base, TROWS_PT)])
            plsc.subcore_barrier()

            def block_body(h, _):
                blk = pl.ds(h * SB, SB)
                pltpu.async_copy(src_hbm.at[p, r, s, blk], src_v, gsem)
                pltpu.async_copy(dst_hbm.at[r, s, blk], dst_v, gsem)
                pltpu.async_copy(w_hbm.at[p, r, s, blk], w_v, gsem)
                pltpu.async_copy(sh_hbm.at[r, s, blk], sh_v, gsem)
                pltpu.make_async_copy(src_hbm.at[p, r, s, blk], src_v,
                                      gsem).wait()
                pltpu.make_async_copy(dst_hbm.at[r, s, blk], dst_v,
                                      gsem).wait()
                pltpu.make_async_copy(w_hbm.at[p, r, s, blk], w_v,
                                      gsem).wait()
                pltpu.make_async_copy(sh_hbm.at[r, s, blk], sh_v,
                                      gsem).wait()

                def chunk_body(j, _):
                    pltpu.async_copy(tab_sh.at[src_v.at[j]], rows_v,
                                     gsem).wait()
                    scale_rows(j)
                    pltpu.sync_copy(rows_v, acc_sh.at[dst_v.at[j]], add=True)
                    return 0
                lax.fori_loop(0, SB, chunk_body, 0)
                return 0
            lax.fori_loop(0, NBLK, block_body, 0)
            plsc.subcore_barrier()

        for q in range(ROWS_PT // ZR):
            base = s * ROWS_PT + q * ZR
            pltpu.sync_copy(acc_sh.at[pl.ds(base, ZR)],
                            out_hbm.at[r, pl.ds(base, ZR)])
        plsc.subcore_barrier()


@functools.cache
def _get_sc_aggregate():
    return pl.kernel(
        _sc_aggregate_body,
        out_type=jax.ShapeDtypeStruct((R, N_PAD, D), jnp.float32),
        mesh=plsc.VectorSubcoreMesh(core_axis_name="c", subcore_axis_name="s",
                                    num_cores=2, num_subcores=NUM_TILES),
        scratch_types=[
            pltpu.VMEM((SB, CHUNK), jnp.int32),
            pltpu.VMEM((SB, CHUNK), jnp.int32),
            pltpu.VMEM((SB, CHUNK), jnp.float32),
            pltpu.VMEM((SB, CHUNK), jnp.int32),
            pltpu.VMEM((CHUNK, D), jnp.float32),
            pltpu.VMEM((ZB, D), jnp.float32),
            pltpu.VMEM_SHARED((TAB_P, D), jnp.float32),
            pltpu.VMEM_SHARED((N_PAD, D), jnp.float32),
            pltpu.SemaphoreType.DMA,
        ],
    )


def _matmul_body(x_ref, w_ref, o_ref):
    o_ref[...] = jnp.dot(x_ref[...], w_ref[...],
                         preferred_element_type=jnp.float32)


def _transform_body(acc_ref, w_ref, o_ref, *, last):
    t = jnp.zeros(o_ref.shape, jnp.float32)
    for r in range(R):
        t = t + lax.dot_general(
            acc_ref[r], w_ref[r],
            dimension_numbers=(((1,), (1,)), ((), ())),
            preferred_element_type=jnp.float32)
    t = jnp.maximum(t, 0.0)
    if last:
        nrm = jnp.sqrt(jnp.sum(t * t, axis=1, keepdims=True))
        t = t / jnp.maximum(nrm, 1e-12)
    o_ref[...] = t


_BN = 1000


def _transform(acc, w, last):
    body = functools.partial(_transform_body, last=last)
    return pl.pallas_call(
        body,
        grid=(N // _BN,),
        in_specs=[
            pl.BlockSpec((R, _BN, D), lambda i: (0, i, 0)),
            pl.BlockSpec((R, D, D), lambda i: (0, 0, 0)),
        ],
        out_specs=pl.BlockSpec((_BN, D), lambda i: (i, 0)),
        out_shape=jax.ShapeDtypeStruct((N, D), jnp.float32),
    )(acc, w)


def kernel(x, edge_index, edge_weight, ent_emb, rel_trans):
    src = edge_index[:, 1, :].reshape(R, NUM_TILES, EPT)
    dst = edge_index[:, 0, :].reshape(R, NUM_TILES, EPT)
    w = edge_weight.reshape(R, NUM_TILES, EPT)
    pad = ((0, 0), (0, 0), (0, EPT_PAD - EPT))
    src = jnp.pad(src, pad).reshape(R, NUM_TILES, NCH, CHUNK)
    dst = jnp.pad(dst, pad).reshape(R, NUM_TILES, NCH, CHUNK)
    w = jnp.pad(w, pad).reshape(R, NUM_TILES, NCH, CHUNK)

    lo = (jnp.arange(NPASS, dtype=jnp.int32) * TP)[:, None, None, None, None]
    src_loc = jnp.clip((src[None] >> 1) - (lo >> 1), 0,
                       TAB_P - 1).astype(jnp.int32)
    in_range = (src[None] >= lo) & (src[None] < lo + TP)
    w_pass = jnp.where(in_range, w[None], 0.0)
    shifts = (src & 1) * 16

    emb = pl.pallas_call(
        _matmul_body,
        grid=(N // _BN,),
        in_specs=[
            pl.BlockSpec((_BN, D), lambda i: (i, 0)),
            pl.BlockSpec((D, D), lambda i: (0, 0)),
        ],
        out_specs=pl.BlockSpec((_BN, D), lambda i: (i, 0)),
        out_shape=jax.ShapeDtypeStruct((N, D), jnp.float32),
    )(x, ent_emb)

    n_layers = rel_trans.shape[0]
    for l in range(n_layers):
        emb_pad = jnp.pad(emb, ((0, N_TAB - N), (0, 0)))
        u16 = lax.bitcast_convert_type(emb_pad.astype(jnp.bfloat16),
                                       jnp.uint16).astype(jnp.uint32)
        tab = u16[0::2] | (u16[1::2] << 16)
        tab = lax.bitcast_convert_type(tab, jnp.float32)
        acc = _get_sc_aggregate()(tab, src_loc, dst, w_pass, shifts)
        emb = _transform(acc, rel_trans[l], last=(l == n_layers - 1))
    return emb
